# baseline (speedup 1.0000x reference)
import functools

import jax
import jax.numpy as jnp
from jax import lax
from jax.experimental import pallas as pl
from jax.experimental.pallas import tpu as pltpu
from jax.experimental.pallas import tpu_sc as plsc

_LANES = 16
_NW = 32
_CHUNK = 256


def _sc_indexed_max_pool(x2, idx_flat, mask_flat, n_k):
    rows, cols = x2.shape
    jk = idx_flat.shape[0]
    idx_flat = jnp.pad(idx_flat, (0, _LANES))
    mask_flat = jnp.pad(mask_flat, (0, _LANES))
    rows_per_w = rows // _NW
    n_chunks = rows_per_w // _CHUNK
    groups = _CHUNK // _LANES

    mesh = plsc.VectorSubcoreMesh(core_axis_name="c", subcore_axis_name="s")

    @functools.partial(
        pl.kernel,
        mesh=mesh,
        compiler_params=pltpu.CompilerParams(needs_layout_passes=False),
        out_type=jax.ShapeDtypeStruct((rows * n_k,), jnp.float32),
        scratch_types=[
            pltpu.VMEM((_CHUNK * cols,), jnp.float32),
            pltpu.VMEM((_CHUNK * n_k,), jnp.float32),
            pltpu.VMEM((jk + _LANES,), jnp.int32),
            pltpu.VMEM((jk + _LANES,), jnp.float32),
        ],
    )
    def run(x_hbm, idx_hbm, mask_hbm, out_hbm, xbuf, obuf, idxv, maskv):
        wid = lax.axis_index("s") * 2 + lax.axis_index("c")
        pltpu.sync_copy(idx_hbm, idxv)
        pltpu.sync_copy(mask_hbm, maskv)
        lane = lax.iota(jnp.int32, _LANES)

        def chunk_body(ci, _):
            base = wid * rows_per_w + ci * _CHUNK
            pltpu.sync_copy(x_hbm.at[pl.ds(base * cols, _CHUNK * cols)], xbuf)

            def group_body(g, _):
                rowvec = lane + g * _LANES
                rowoff = rowvec * cols
                rowout = rowvec * n_k

                def j_body(t, accs):
                    vidx = idxv[pl.ds(t * n_k, _LANES)]
                    vmask = maskv[pl.ds(t * n_k, _LANES)]
                    new = []
                    for kk in range(n_k):
                        col = plsc.load_gather(xbuf, [rowoff + vidx[kk]])
                        new.append(jnp.maximum(accs[kk], col * vmask[kk]))
                    return tuple(new)

                init = tuple(
                    jnp.full((_LANES,), -jnp.inf, jnp.float32) for _ in range(n_k)
                )
                accs = lax.fori_loop(0, jk // n_k, j_body, init)
                for kk in range(n_k):
                    plsc.store_scatter(obuf, [rowout + kk], accs[kk])
                return 0

            lax.fori_loop(0, groups, group_body, 0)
            pltpu.sync_copy(obuf, out_hbm.at[pl.ds(base * n_k, _CHUNK * n_k)])
            return 0

        lax.fori_loop(0, n_chunks, chunk_body, 0)

    return run(x2.reshape(-1), idx_flat, mask_flat)


def kernel(input_images, indices, mask):
    b, f, c = input_images.shape
    n_k = indices.shape[-1]
    x2 = input_images.reshape(b * f, c)
    idx_flat = indices.reshape(-1).astype(jnp.int32)
    mask_flat = mask.reshape(-1).astype(jnp.float32)
    out = _sc_indexed_max_pool(x2, idx_flat, mask_flat, n_k)
    return out.reshape(b, f, n_k)


# speedup vs baseline: 2.1866x; 2.1866x over previous
import functools

import jax
import jax.numpy as jnp
from jax import lax
from jax.experimental import pallas as pl
from jax.experimental.pallas import tpu as pltpu
from jax.experimental.pallas import tpu_sc as plsc

_LANES = 16
_NW = 32
_CHUNK = 128


def _sc_indexed_max_pool(x_flat, adj_idx, zfix, n_k, rows, cols):
    jk = adj_idx.shape[0]
    n_vecs = jk // _LANES
    rows_per_w = rows // _NW
    n_chunks = rows_per_w // _CHUNK

    mesh = plsc.VectorSubcoreMesh(core_axis_name="c", subcore_axis_name="s")

    @functools.partial(
        pl.kernel,
        mesh=mesh,
        compiler_params=pltpu.CompilerParams(needs_layout_passes=False),
        out_type=jax.ShapeDtypeStruct((rows * n_k,), jnp.float32),
        scratch_types=[
            pltpu.VMEM((_CHUNK * cols,), jnp.float32),
            pltpu.VMEM((_CHUNK * cols,), jnp.float32),
            pltpu.VMEM((_CHUNK * n_k,), jnp.float32),
            pltpu.VMEM((_CHUNK * n_k,), jnp.float32),
            pltpu.VMEM((jk,), jnp.int32),
            pltpu.VMEM((_LANES,), jnp.float32),
            pltpu.SemaphoreType.DMA,
            pltpu.SemaphoreType.DMA,
            pltpu.SemaphoreType.DMA,
            pltpu.SemaphoreType.DMA,
        ],
    )
    def run(x_hbm, adj_hbm, zfix_hbm, out_hbm,
            xb0, xb1, ob0, ob1, adjr, zr, si0, si1, so0, so1):
        wid = lax.axis_index("s") * 2 + lax.axis_index("c")
        pltpu.sync_copy(adj_hbm, adjr)
        pltpu.sync_copy(zfix_hbm, zr)
        lane = lax.iota(jnp.int32, _LANES)
        zfix = zr[...]
        lmask = lane < n_k
        rot8 = (lane + 8) & 15
        rot4 = (lane + 4) & 15
        adjvecs = [adjr[pl.ds(c * _LANES, _LANES)] for c in range(n_vecs)]
        row0 = wid * rows_per_w

        def vrot(v, perm):
            return lax.gather(
                v,
                perm.reshape(_LANES, 1),
                lax.GatherDimensionNumbers(
                    offset_dims=(),
                    collapsed_slice_dims=(0,),
                    start_index_map=(0,),
                ),
                (1,),
                mode=lax.GatherScatterMode.PROMISE_IN_BOUNDS,
            )

        def start_in(ci):
            b = ci % 2
            return pltpu.async_copy(
                x_hbm.at[pl.ds((row0 + ci * _CHUNK) * cols, _CHUNK * cols)],
                (xb0, xb1)[b],
                (si0, si1)[b],
            )

        in_handles = [None] * n_chunks
        out_handles = [None] * n_chunks
        in_handles[0] = start_in(0)
        for ci in range(n_chunks):
            b = ci % 2
            xbuf, obuf = (xb0, xb1)[b], (ob0, ob1)[b]
            in_handles[ci].wait()
            if ci + 1 < n_chunks:
                in_handles[ci + 1] = start_in(ci + 1)
            if ci >= 2:
                out_handles[ci - 2].wait()

            def row_body(r, _, xbuf=xbuf, obuf=obuf):
                rbase = jnp.full((_LANES,), r * cols, jnp.int32)
                acc = zfix
                for c in range(n_vecs):
                    g = plsc.load_gather(xbuf, [adjvecs[c] + rbase])
                    acc = jnp.maximum(acc, g)
                acc = jnp.maximum(acc, vrot(acc, rot8))
                acc = jnp.maximum(acc, vrot(acc, rot4))
                plsc.store_scatter(obuf, [lane + r * n_k], acc, mask=lmask)
                return 0

            lax.fori_loop(0, _CHUNK, row_body, 0, unroll=2)
            out_handles[ci] = pltpu.async_copy(
                obuf,
                out_hbm.at[pl.ds((row0 + ci * _CHUNK) * n_k, _CHUNK * n_k)],
                (so0, so1)[b],
            )
        out_handles[n_chunks - 2].wait()
        out_handles[n_chunks - 1].wait()

    return run(x_flat, adj_idx, zfix)


def kernel(input_images, indices, mask):
    b, f, c = input_images.shape
    n_k = indices.shape[-1]
    rows = b * f
    idx2 = indices.reshape(-1, n_k).astype(jnp.int32)
    mask2 = mask.reshape(-1, n_k)
    first_live = jnp.argmax(mask2, axis=0)
    live_idx = jnp.take_along_axis(idx2, first_live[None, :], axis=0)
    adj = jnp.where(mask2 > 0, idx2, live_idx).reshape(-1)
    has_zero = jnp.any(mask2.reshape(-1, _LANES) == 0, axis=0)
    zfix = jnp.where(has_zero, 0.0, -jnp.inf).astype(jnp.float32)
    out = _sc_indexed_max_pool(input_images.reshape(-1), adj, zfix,
                               n_k, rows, c)
    return out.reshape(b, f, n_k)


# speedup vs baseline: 3.0210x; 1.3816x over previous
import functools

import jax
import jax.numpy as jnp
from jax import lax
from jax.experimental import pallas as pl
from jax.experimental.pallas import tpu as pltpu
from jax.experimental.pallas import tpu_sc as plsc

_LANES = 16
_NW = 32
_CHUNK = 128


def _sc_indexed_max_pool(x_flat, adj_idx, zfix, n_k, rows, cols):
    jk = adj_idx.shape[0]
    n_vecs = jk // _LANES
    rows_per_w = rows // _NW
    n_chunks = rows_per_w // _CHUNK

    mesh = plsc.VectorSubcoreMesh(core_axis_name="c", subcore_axis_name="s")

    @functools.partial(
        pl.kernel,
        mesh=mesh,
        compiler_params=pltpu.CompilerParams(
            needs_layout_passes=False, use_tc_tiling_on_sc=True
        ),
        out_type=jax.ShapeDtypeStruct((rows * n_k,), jnp.float32),
        scratch_types=[
            pltpu.VMEM((_CHUNK, cols), jnp.float32),
            pltpu.VMEM((_CHUNK, cols), jnp.float32),
            pltpu.VMEM((_CHUNK * n_k,), jnp.float32),
            pltpu.VMEM((_CHUNK * n_k,), jnp.float32),
            pltpu.VMEM((jk,), jnp.int32),
            pltpu.VMEM((_LANES,), jnp.float32),
            pltpu.SemaphoreType.DMA,
            pltpu.SemaphoreType.DMA,
            pltpu.SemaphoreType.DMA,
            pltpu.SemaphoreType.DMA,
        ],
    )
    def run(x_hbm, adj_hbm, zfix_hbm, out_hbm,
            xb0, xb1, ob0, ob1, adjr, zr, si0, si1, so0, so1):
        wid = lax.axis_index("s") * 2 + lax.axis_index("c")
        pltpu.sync_copy(adj_hbm, adjr)
        pltpu.sync_copy(zfix_hbm, zr)
        lane = lax.iota(jnp.int32, _LANES)
        zfix = zr[...]
        lmask = lane < n_k
        rot8 = (lane + 8) & 15
        rot4 = (lane + 4) & 15
        adjvecs = [adjr[pl.ds(c * _LANES, _LANES)] for c in range(n_vecs)]
        row0 = wid * rows_per_w

        def vrot(v, perm):
            return lax.gather(
                v,
                perm.reshape(_LANES, 1),
                lax.GatherDimensionNumbers(
                    offset_dims=(),
                    collapsed_slice_dims=(0,),
                    start_index_map=(0,),
                ),
                (1,),
                mode=lax.GatherScatterMode.PROMISE_IN_BOUNDS,
            )

        def start_in(ci):
            b = ci % 2
            return pltpu.async_copy(
                x_hbm.at[pl.ds(row0 + ci * _CHUNK, _CHUNK), :],
                (xb0, xb1)[b],
                (si0, si1)[b],
            )

        in_handles = [None] * n_chunks
        out_handles = [None] * n_chunks
        in_handles[0] = start_in(0)
        for ci in range(n_chunks):
            b = ci % 2
            xbuf, obuf = (xb0, xb1)[b], (ob0, ob1)[b]
            in_handles[ci].wait()
            if ci + 1 < n_chunks:
                in_handles[ci + 1] = start_in(ci + 1)
            if ci >= 2:
                out_handles[ci - 2].wait()

            def row_body(r, _, xbuf=xbuf, obuf=obuf):
                rvec = jnp.full((_LANES,), r, jnp.int32)
                acc = zfix
                for c in range(n_vecs):
                    g = plsc.load_gather(xbuf, [rvec, adjvecs[c]])
                    acc = jnp.maximum(acc, g)
                acc = jnp.maximum(acc, vrot(acc, rot8))
                acc = jnp.maximum(acc, vrot(acc, rot4))
                plsc.store_scatter(obuf, [lane + r * n_k], acc, mask=lmask)
                return 0

            lax.fori_loop(0, _CHUNK, row_body, 0, unroll=2)
            out_handles[ci] = pltpu.async_copy(
                obuf,
                out_hbm.at[pl.ds((row0 + ci * _CHUNK) * n_k, _CHUNK * n_k)],
                (so0, so1)[b],
            )
        out_handles[n_chunks - 2].wait()
        out_handles[n_chunks - 1].wait()

    return run(x_flat, adj_idx, zfix)


def kernel(input_images, indices, mask):
    b, f, c = input_images.shape
    n_k = indices.shape[-1]
    rows = b * f
    idx2 = indices.reshape(-1, n_k).astype(jnp.int32)
    mask2 = mask.reshape(-1, n_k)
    first_live = jnp.argmax(mask2, axis=0)
    live_idx = jnp.take_along_axis(idx2, first_live[None, :], axis=0)
    adj = jnp.where(mask2 > 0, idx2, live_idx).reshape(-1)
    has_zero = jnp.any(mask2.reshape(-1, _LANES) == 0, axis=0)
    zfix = jnp.where(has_zero, 0.0, -jnp.inf).astype(jnp.float32)
    out = _sc_indexed_max_pool(input_images.reshape(rows, c), adj, zfix,
                               n_k, rows, c)
    return out.reshape(b, f, n_k)
